# Initial kernel scaffold; baseline (speedup 1.0000x reference)
import jax
import jax.numpy as jnp
from jax.experimental import pallas as pl


def kernel(rs, params, coords, senders_same, receivers_same, senders_anti, receivers_anti, senders_ne, receivers_ne):
    raise NotImplementedError("write your pallas kernel here")


# speedup vs baseline: 3.2552x; 3.2552x over previous
import functools

import numpy as np
import jax
import jax.numpy as jnp
from jax import lax
from jax.experimental import pallas as pl
from jax.experimental.pallas import tpu as pltpu
from jax.experimental.pallas import tpu_sc as plsc

CUTOFF = 10.0
N_CORES = 2
N_SUB = 16
TILES = N_CORES * N_SUB
LANES = 16


def _mesh():
    return plsc.VectorSubcoreMesh(core_axis_name="c", subcore_axis_name="s")

_SC_PARAMS = pltpu.CompilerParams(needs_layout_passes=False)


def _make_ssq3(ee_words, ne_words, ep):
    per_tile = ep // TILES
    steps = per_tile // LANES

    @functools.partial(
        pl.kernel,
        out_type=[jax.ShapeDtypeStruct((ep,), jnp.float32)] * 3,
        mesh=_mesh(),
        compiler_params=_SC_PARAMS,
        scratch_types=[
            pltpu.VMEM((ee_words,), jnp.float32),
            pltpu.VMEM((ne_words,), jnp.float32),
            pltpu.VMEM((per_tile,), jnp.int32),
            pltpu.VMEM((per_tile,), jnp.int32),
            pltpu.VMEM((per_tile,), jnp.float32),
        ],
    )
    def k(rs_hbm, co_hbm, ss_hbm, rr_s_hbm, sa_hbm, ra_hbm, sn_hbm, rn_hbm,
          os_hbm, oa_hbm, on_hbm, rs_v, co_v, s_v, r_v, o_v):
        tid = lax.axis_index("c") * N_SUB + lax.axis_index("s")
        base = tid * per_tile
        pltpu.sync_copy(rs_hbm, rs_v)
        pltpu.sync_copy(co_hbm, co_v)

        def phase(stab_v, s_hbm, r_hbm, out_hbm):
            pltpu.sync_copy(s_hbm.at[pl.ds(base, per_tile)], s_v)
            pltpu.sync_copy(r_hbm.at[pl.ds(base, per_tile)], r_v)

            def body(i, _):
                off = i * LANES
                s16 = s_v[pl.ds(off, LANES)] * 4
                r16 = r_v[pl.ds(off, LANES)] * 4
                acc = jnp.zeros((LANES,), jnp.float32)
                for c in range(3):
                    a = plsc.load_gather(stab_v, [s16 + c])
                    b = plsc.load_gather(rs_v, [r16 + c])
                    d = b - a
                    acc = acc + d * d
                o_v[pl.ds(off, LANES)] = acc
                return 0

            lax.fori_loop(0, steps, body, 0)
            pltpu.sync_copy(o_v, out_hbm.at[pl.ds(base, per_tile)])

        phase(rs_v, ss_hbm, rr_s_hbm, os_hbm)
        phase(rs_v, sa_hbm, ra_hbm, oa_hbm)
        phase(co_v, sn_hbm, rn_hbm, on_hbm)

    return k


def _make_wmlp(kind, ep, n_layers, blk=2048):
    grid = ep // blk

    def body(*refs):
        if kind == "ne":
            ssq_ref, rcv_ref, sid_ref, y_ref, mus_ref, sg2_ref, w1_ref, b1_ref, w2_ref, out_ref = refs
        else:
            ssq_ref, rcv_ref, x_ref, h0_ref, mus_ref, sg2_ref, w1_ref, b1_ref, w2_ref, out_ref = refs
        bit = jnp.bitwise_and(rcv_ref[...], 1).astype(jnp.float32)
        left, right = 1.0 - bit, bit
        d = jnp.sqrt(ssq_ref[...] + 1e-12)
        env = d * d * jnp.exp(-d)
        mus = mus_ref[...]
        sg2 = sg2_ref[...]
        basis = env * jnp.exp(-((d - mus) ** 2) / sg2)
        if kind == "ne":
            sid = sid_ref[...]
            onehot = (sid == lax.broadcasted_iota(jnp.int32, (1, 16), 1)).astype(jnp.float32)
            rows = jnp.dot(onehot, y_ref[...], preferred_element_type=jnp.float32)
        else:
            hx0 = jnp.dot(x_ref[...], h0_ref[...], preferred_element_type=jnp.float32)
        for l in range(n_layers):
            h1 = jnp.dot(basis, w1_ref[l], preferred_element_type=jnp.float32) + b1_ref[l]
            a = jnp.logaddexp(h1, 0.0) - np.float32(np.log(2.0))
            w = jnp.dot(a, w2_ref[l], preferred_element_type=jnp.float32)
            if kind == "ne":
                w = w * rows
            elif l == 0:
                w = w * hx0
            out_ref[l] = jnp.concatenate([w * left, w * right], axis=-1)

    full = lambda *shape: pl.BlockSpec(shape, lambda i: (0,) * len(shape))
    in_specs = [pl.BlockSpec((blk, 1), lambda i: (i, 0)),
                pl.BlockSpec((blk, 1), lambda i: (i, 0))]
    if kind == "ne":
        in_specs += [pl.BlockSpec((blk, 1), lambda i: (i, 0)), full(16, 64)]
    else:
        in_specs += [full(1, 128), full(128, 64)]
    in_specs += [full(1, 32), full(1, 32), full(n_layers, 32, 45),
                 full(n_layers, 1, 45), full(n_layers, 45, 64)]

    return pl.pallas_call(
        body,
        grid=(grid,),
        in_specs=in_specs,
        out_specs=pl.BlockSpec((n_layers, blk, 128), lambda i: (0, i, 0)),
        out_shape=jax.ShapeDtypeStruct((n_layers, ep, 128), jnp.float32),
    )


def _make_seg3(gather, lstat, table_rows, ep, zh):
    per_tile = ep // TILES
    ch = 64
    nch = per_tile // ch
    rows_pt = zh // N_SUB
    NB = 3
    NG = 2
    PD = 2
    STEP = 6

    scratch = [
        pltpu.VMEM((nch, ch), jnp.int32),
        [pltpu.VMEM((ch, 128), jnp.float32) for _ in range(NB)],
        pltpu.VMEM_SHARED((zh, 128), jnp.float32),
        [pltpu.SemaphoreType.DMA for _ in range(NB)],
        [pltpu.SemaphoreType.DMA for _ in range(NB)],
    ]
    if gather:
        scratch += [
            pltpu.VMEM((per_tile,), jnp.int32),
            [pltpu.VMEM((ch, 128), jnp.float32) for _ in range(NG)],
            [pltpu.SemaphoreType.DMA for _ in range(NG)],
        ]

    def k(*refs):
        if gather:
            (ws_hbm, wa_hbm, wn_hbm, ss_hbm, sa_hbm, rr_hbm, ra_hbm, rn_hbm,
             table_hbm, zero_hbm, zs_hbm, za_hbm, zn_hbm,
             r_v, w_ring, z_sh, sem_w, sem_s, s_v, g_ring, sem_g) = refs
        else:
            (ws_hbm, wa_hbm, wn_hbm, rr_hbm, ra_hbm, rn_hbm, zero_hbm,
             zs_hbm, za_hbm, zn_hbm,
             r_v, w_ring, z_sh, sem_w, sem_s) = refs
        cid = lax.axis_index("c")
        sid = lax.axis_index("s")
        tid = cid * N_SUB + sid
        base = tid * per_tile

        def phase(w_hbm, s_hbm, r_hbm, out_hbm, use_g):
            pltpu.sync_copy(zero_hbm.at[pl.ds(sid * rows_pt, rows_pt)],
                            z_sh.at[pl.ds(sid * rows_pt, rows_pt)])
            pltpu.sync_copy(r_hbm.at[tid], r_v)
            if use_g:
                pltpu.sync_copy(s_hbm.at[pl.ds(base, per_tile)], s_v)
            plsc.subcore_barrier()

            def start_load(kk, b, bg):
                pltpu.async_copy(w_hbm.at[lstat, pl.ds(base + kk * ch, ch)],
                                 w_ring[b], sem_w[b])
                if use_g:
                    pltpu.async_copy(table_hbm.at[s_v.at[pl.ds(kk * ch, ch)]],
                                     g_ring[bg], sem_g[bg])

            for c in range(PD):
                start_load(c, c % NB, c % NG)

            def chunk_body(kk, b, bg):
                pltpu.make_async_copy(
                    w_hbm.at[lstat, pl.ds(base, ch)], w_ring[b], sem_w[b]).wait()
                if use_g:
                    pltpu.make_async_copy(
                        table_hbm.at[s_v.at[pl.ds(0, ch)]], g_ring[bg], sem_g[bg]).wait()

                    def mrow(j, _2):
                        for c4 in range(8):
                            sl = pl.ds(c4 * LANES, LANES)
                            w_ring[b][j, sl] = w_ring[b][j, sl] * g_ring[bg][j, sl]
                        return 0

                    lax.fori_loop(0, ch, mrow, 0, unroll=4)
                pltpu.async_copy(w_ring[b], z_sh.at[r_v.at[kk]], sem_s[b], add=True)
                b2 = (b + PD) % NB

                def wait_prev_scatter():
                    pltpu.make_async_copy(
                        w_ring[b2], z_sh.at[r_v.at[0]], sem_s[b2]).wait()

                def prefetch():
                    start_load(kk + PD, b2, bg)

                if isinstance(kk, int):
                    if kk + PD >= NB:
                        wait_prev_scatter()
                    if kk + PD < nch:
                        prefetch()
                else:
                    pl.when(kk + PD >= NB)(wait_prev_scatter)
                    pl.when(kk + PD < nch)(prefetch)

            def group(g, _):
                for i in range(STEP):
                    chunk_body(g * STEP + i, i % NB, i % NG)
                return 0

            lax.fori_loop(0, nch // STEP, group, 0)
            for c in range((nch // STEP) * STEP, nch):
                chunk_body(c, c % NB, c % NG)
            for c in range(nch - (NB - PD), nch):
                b = c % NB
                pltpu.make_async_copy(w_ring[b], z_sh.at[r_v.at[0]], sem_s[b]).wait()
            plsc.subcore_barrier()
            pltpu.sync_copy(z_sh.at[pl.ds(sid * rows_pt, rows_pt)],
                            out_hbm.at[cid, pl.ds(sid * rows_pt, rows_pt)])

        phase(ws_hbm, ss_hbm if gather else None, rr_hbm, zs_hbm, gather)
        phase(wa_hbm, sa_hbm if gather else None, ra_hbm, za_hbm, gather)
        phase(wn_hbm, None, rn_hbm, zn_hbm, False)

    return functools.partial(
        pl.kernel,
        out_type=[jax.ShapeDtypeStruct((N_CORES, zh, 128), jnp.float32)] * 3,
        mesh=_mesh(),
        compiler_params=_SC_PARAMS,
        scratch_types=scratch,
    )(k)


def _make_update(first, n, zr, blkd=1000):
    grid = n // blkd

    def body(e_ref, zs_ref, za_ref, zn_ref, g_ref, hn_ref, eo_ref, hx_ref):
        zcat = jnp.concatenate(
            [zs_ref[0], za_ref[0], zn_ref[0], zs_ref[1], za_ref[1], zn_ref[1]],
            axis=-1)
        acc = jnp.dot(zcat, g_ref[...], preferred_element_type=jnp.float32)
        e = e_ref[...] + acc
        eo_ref[...] = e
        hx = jnp.dot(e, hn_ref[...], preferred_element_type=jnp.float32)
        hx_ref[...] = jnp.concatenate([hx, hx], axis=-1)

    zspec = pl.BlockSpec((N_CORES, blkd, 64), lambda i: (0, i, 0))
    e_spec = (pl.BlockSpec((1, 128), lambda i: (0, 0)) if first
              else pl.BlockSpec((blkd, 128), lambda i: (i, 0)))
    return pl.pallas_call(
        body,
        grid=(grid,),
        in_specs=[e_spec, zspec, zspec, zspec,
                  pl.BlockSpec((384, 128), lambda i: (0, 0)),
                  pl.BlockSpec((128, 64), lambda i: (0, 0))],
        out_specs=[pl.BlockSpec((blkd, 128), lambda i: (i, 0)),
                   pl.BlockSpec((blkd, 128), lambda i: (i, 0))],
        out_shape=[jax.ShapeDtypeStruct((n, 128), jnp.float32),
                   jax.ShapeDtypeStruct((n, 128), jnp.float32)],
    )


def kernel(rs, params, coords, senders_same, receivers_same,
           senders_anti, receivers_anti, senders_ne, receivers_ne):
    n = rs.shape[0]
    e = senders_same.shape[0]
    layers = params['layers']
    n_layers = len(layers)
    dfd = layers[0]['w_same']['W1'].shape[0]

    ch = 64
    per_tile = ((e + TILES * ch - 1) // (TILES * ch)) * ch
    ep = per_tile * TILES
    nch = per_tile // ch
    zr = ((n + 1 + 255) // 256) * 256
    zh = zr // 2
    dummy = n

    def pad_idx(x, val):
        return jnp.concatenate([x, jnp.full((ep - e,), val, jnp.int32)])

    s_s = pad_idx(senders_same, 0)
    r_s = pad_idx(receivers_same, dummy)
    s_a = pad_idx(senders_anti, 0)
    r_a = pad_idx(receivers_anti, dummy)
    s_n = pad_idx(senders_ne, 0)
    r_n = pad_idx(receivers_ne, dummy)

    def flat_table(x):
        f = jnp.pad(x, ((0, 0), (0, 1))).reshape(-1)
        pad = (-f.shape[0]) % 128
        return jnp.pad(f, (0, pad))

    rs4 = flat_table(rs)
    co4 = flat_table(coords)

    ssq_s, ssq_a, ssq_n = _make_ssq3(rs4.shape[0], co4.shape[0], ep)(
        rs4, co4, s_s, r_s, s_a, r_a, s_n, r_n)

    delta = 1.0 / (2 * dfd)
    qs = np.linspace(delta, 1.0 - delta, dfd)
    mus = jnp.asarray((CUTOFF * qs ** 2)[None, :], jnp.float32)
    sg2 = jnp.asarray((((1.0 + CUTOFF * qs) / 7.0) ** 2)[None, :], jnp.float32)

    def stack(lbl, name):
        return jnp.stack([lp['w_' + lbl][name] for lp in layers])

    args = {}
    for lbl in ('same', 'anti', 'ne'):
        args[lbl] = (stack(lbl, 'W1'), stack(lbl, 'b1').reshape(n_layers, 1, -1),
                     stack(lbl, 'W2'))

    wmlp_pair = _make_wmlp("pair", ep, n_layers)
    w_same = wmlp_pair(ssq_s.reshape(ep, 1), r_s.reshape(ep, 1),
                       params['X'], layers[0]['h'], mus, sg2, *args['same'])
    w_anti = wmlp_pair(ssq_a.reshape(ep, 1), r_a.reshape(ep, 1),
                       params['X'], layers[0]['h'], mus, sg2, *args['anti'])
    weh_ne = _make_wmlp("ne", ep, n_layers)(
        ssq_n.reshape(ep, 1), r_n.reshape(ep, 1), s_n.reshape(ep, 1),
        params['Y'], mus, sg2, *args['ne'])

    r_s3 = (r_s >> 1).reshape(TILES, nch, ch)
    r_a3 = (r_a >> 1).reshape(TILES, nch, ch)
    r_n3 = (r_n >> 1).reshape(TILES, nch, ch)
    zeros = jnp.zeros((zh, 128), jnp.float32)

    seg0 = _make_seg3(False, 0, 0, ep, zh)
    seg_g = [None] + [_make_seg3(True, l, n, ep, zh) for l in range(1, n_layers)]

    elec = params['X']
    hx = None
    for l in range(n_layers):
        if l == 0:
            z_s, z_a, z_n = seg0(w_same, w_anti, weh_ne, r_s3, r_a3, r_n3, zeros)
        else:
            z_s, z_a, z_n = seg_g[l](w_same, w_anti, weh_ne, s_s, s_a,
                                     r_s3, r_a3, r_n3, hx, zeros)
        z_s, z_a, z_n = (z.reshape(N_CORES, zr, 64) for z in (z_s, z_a, z_n))
        g_cat = jnp.concatenate(
            [layers[l]['g_same'], layers[l]['g_anti'], layers[l]['g_ne']] * 2, axis=0)
        h_next = layers[(l + 1) % n_layers]['h']
        elec, hx = _make_update(l == 0, n, zr)(elec, z_s, z_a, z_n, g_cat, h_next)
    return elec
